# baseline (speedup 1.0000x reference)
import functools

import jax
import jax.numpy as jnp
from jax.experimental import pallas as pl
from jax.experimental.pallas import tpu as pltpu

B = 128
S = 256
DIM = 768
EMB = 256
K = 8192
R = B * 4
EPS = 1e-12


def _mm_kernel(x_ref, w_ref, b_ref, o_ref, *, relu, out_bf16):
    acc = jnp.dot(x_ref[...], w_ref[...], preferred_element_type=jnp.float32)
    acc = acc + b_ref[...]
    if relu:
        acc = jnp.maximum(acc, 0.0)
    if out_bf16:
        acc = acc.astype(jnp.bfloat16)
    o_ref[...] = acc


def _matmul(x, w, b, relu=False, out_bf16=False, tm=1024):
    m, kc = x.shape
    n = w.shape[1]
    tm = min(tm, m)
    odt = jnp.bfloat16 if out_bf16 else jnp.float32
    return pl.pallas_call(
        functools.partial(_mm_kernel, relu=relu, out_bf16=out_bf16),
        grid=(m // tm,),
        in_specs=[
            pl.BlockSpec((tm, kc), lambda i: (i, 0)),
            pl.BlockSpec((kc, n), lambda i: (0, 0)),
            pl.BlockSpec((1, n), lambda i: (0, 0)),
        ],
        out_specs=pl.BlockSpec((tm, n), lambda i: (i, 0)),
        out_shape=jax.ShapeDtypeStruct((m, n), odt),
    )(x, w, b.reshape(1, n))


def _im2col(x, stride, oh, pad):
    if pad:
        x = jnp.pad(x, ((0, 0), (pad, pad), (pad, pad), (0, 0)))
    cols = []
    for kh in range(3):
        for kw in range(3):
            cols.append(x[:, kh:kh + stride * oh:stride,
                          kw:kw + stride * oh:stride, :])
    p = jnp.concatenate(cols, axis=-1)
    return p.reshape(-1, p.shape[-1])


def _vq_kernel(zf_ref, zl_ref, cbt_ref, noise_ref, idx_ref, q_ref,
               min_ref, arg_ref, *, kt, nkt):
    k = pl.program_id(0)
    x = zl_ref[...] - zf_ref[...]
    cbt = cbt_ref[...]
    cn = jnp.sum(cbt * cbt, axis=0, keepdims=True)
    scores = cn - 2.0 * jnp.dot(x.astype(jnp.bfloat16),
                                cbt.astype(jnp.bfloat16),
                                preferred_element_type=jnp.float32)
    lmin = jnp.min(scores, axis=1, keepdims=True)
    iota = jax.lax.broadcasted_iota(jnp.int32, scores.shape, 1) + k * kt
    larg = jnp.min(jnp.where(scores == lmin, iota, jnp.int32(2 ** 30)),
                   axis=1, keepdims=True)

    @pl.when(k == 0)
    def _():
        min_ref[...] = lmin
        arg_ref[...] = larg

    @pl.when(k > 0)
    def _():
        better = lmin < min_ref[...]
        min_ref[...] = jnp.where(better, lmin, min_ref[...])
        arg_ref[...] = jnp.where(better, larg, arg_ref[...])

    @pl.when(k == nkt - 1)
    def _():
        xn = jnp.sum(x * x, axis=1, keepdims=True)
        norm_res = jnp.sqrt(jnp.maximum(min_ref[...] + xn, 0.0))
        noise = noise_ref[...]
        norm_rand = jnp.sqrt(jnp.sum(noise * noise, axis=1, keepdims=True))
        q_ref[...] = x + (norm_res / norm_rand + EPS) * noise
        idx_ref[...] = arg_ref[...]


def _vq(zf, zl, cbt, noise, kt=2048):
    nkt = K // kt
    return pl.pallas_call(
        functools.partial(_vq_kernel, kt=kt, nkt=nkt),
        grid=(nkt,),
        in_specs=[
            pl.BlockSpec((R, EMB), lambda k: (0, 0)),
            pl.BlockSpec((R, EMB), lambda k: (0, 0)),
            pl.BlockSpec((EMB, kt), lambda k: (0, k)),
            pl.BlockSpec((R, EMB), lambda k: (0, 0)),
        ],
        out_specs=[
            pl.BlockSpec((R, 1), lambda k: (0, 0)),
            pl.BlockSpec((R, EMB), lambda k: (0, 0)),
        ],
        out_shape=[
            jax.ShapeDtypeStruct((R, 1), jnp.int32),
            jax.ShapeDtypeStruct((R, EMB), jnp.float32),
        ],
        scratch_shapes=[
            pltpu.VMEM((R, 1), jnp.float32),
            pltpu.VMEM((R, 1), jnp.int32),
        ],
    )(zf, zl, cbt, noise)


def _perp_kernel(idx_ref, o_ref, acc_ref, *, kt, nkt):
    k = pl.program_id(0)
    idx = idx_ref[...]
    iota = jax.lax.broadcasted_iota(jnp.int32, (R, kt), 1) + k * kt
    counts = jnp.sum((idx == iota).astype(jnp.float32), axis=0, keepdims=True)
    p = counts / R
    ent = jnp.sum(p * jnp.log(p + 1e-10), axis=1, keepdims=True)

    @pl.when(k == 0)
    def _():
        acc_ref[...] = ent

    @pl.when(k > 0)
    def _():
        acc_ref[...] = acc_ref[...] + ent

    @pl.when(k == nkt - 1)
    def _():
        o_ref[...] = jnp.exp(-acc_ref[...])


def _perplexity(idx, kt=2048):
    nkt = K // kt
    return pl.pallas_call(
        functools.partial(_perp_kernel, kt=kt, nkt=nkt),
        grid=(nkt,),
        in_specs=[pl.BlockSpec((R, 1), lambda k: (0, 0))],
        out_specs=pl.BlockSpec((1, 1), lambda k: (0, 0)),
        out_shape=jax.ShapeDtypeStruct((1, 1), jnp.float32),
        scratch_shapes=[pltpu.VMEM((1, 1), jnp.float32)],
    )(idx)


def kernel(input_data_first, input_data_last, codebooks, W_in, b_in,
           conv1_w, conv1_b, conv2_w, conv2_b, conv3_w, conv3_b,
           W_out, b_out, noise):
    bf = jnp.bfloat16
    xin = jnp.concatenate([input_data_first.reshape(B * S, DIM),
                           input_data_last.reshape(B * S, DIM)],
                          axis=0).astype(bf)
    h = _matmul(xin, W_in.astype(bf), b_in, out_bf16=True,
                tm=2048).reshape(2 * B, 16, 16, EMB)
    c1 = _matmul(_im2col(h, 2, 8, 1), conv1_w.reshape(9 * EMB, EMB).astype(bf),
                 conv1_b, relu=True, out_bf16=True,
                 tm=1024).reshape(2 * B, 8, 8, EMB)
    c2 = _matmul(_im2col(c1, 2, 4, 1), conv2_w.reshape(9 * EMB, EMB).astype(bf),
                 conv2_b, relu=True, out_bf16=True,
                 tm=1024).reshape(2 * B, 4, 4, EMB)
    z = _matmul(_im2col(c2, 1, 2, 0), conv3_w.reshape(9 * EMB, EMB).astype(bf),
                conv3_b, tm=1024).reshape(2 * R, EMB)
    zf, zl = z[:R], z[R:]
    idx, q = _vq(zf, zl, codebooks.T, noise)
    perp = _perplexity(idx)
    q2 = q.reshape(B, EMB, 4).transpose(0, 2, 1).reshape(R, EMB).astype(bf)
    out = _matmul(q2, W_out.astype(bf), b_out, tm=512).reshape(B, 4, DIM)
    return out, perp.reshape(()), idx.reshape(R)


# speedup vs baseline: 42.4211x; 42.4211x over previous
import functools

import jax
import jax.numpy as jnp
from jax.experimental import pallas as pl
from jax.experimental.pallas import tpu as pltpu

B = 128
S = 256
DIM = 768
EMB = 256
K = 8192
R = B * 4
EPS = 1e-12
BF = jnp.bfloat16
F32 = jnp.float32


def _proj_kernel(x_ref, w_ref, b_ref, o_ref):
    acc = jnp.dot(x_ref[...].astype(BF), w_ref[...],
                  preferred_element_type=F32)
    o_ref[...] = (acc + b_ref[...]).astype(BF)


def _proj(x, w_bf, b, tm=2048):
    m = x.shape[0]
    return pl.pallas_call(
        _proj_kernel,
        grid=(m // tm,),
        in_specs=[
            pl.BlockSpec((tm, DIM), lambda i: (i, 0)),
            pl.BlockSpec((DIM, EMB), lambda i: (0, 0)),
            pl.BlockSpec((1, EMB), lambda i: (0, 0)),
        ],
        out_specs=pl.BlockSpec((tm, EMB), lambda i: (i, 0)),
        out_shape=jax.ShapeDtypeStruct((m, EMB), BF),
    )(x, w_bf, b.reshape(1, EMB))


def _conv_s2_kernel(x_ref, w_ref, b_ref, o_ref, *, nt, oh, relu, out_f32):
    ph = oh + 1
    x = x_ref[...]
    xp = jnp.pad(x, ((0, 0), (1, 1), (1, 1), (0, 0)))
    xr = xp.reshape(nt, ph, 2, ph, 2, EMB)
    acc = b_ref[...].astype(F32)
    for kh in range(3):
        for kw in range(3):
            xs = xr[:, kh // 2:kh // 2 + oh, kh % 2,
                    kw // 2:kw // 2 + oh, kw % 2, :]
            a = xs.reshape(nt * oh * oh, EMB)
            j = kh * 3 + kw
            wj = w_ref[j * EMB:(j + 1) * EMB, :]
            acc = acc + jnp.dot(a, wj, preferred_element_type=F32)
    if relu:
        acc = jnp.maximum(acc, 0.0)
    o_ref[...] = acc if out_f32 else acc.astype(BF)


def _conv_s2(x, w_bf, b, oh, relu, nt, out_f32=False):
    n = x.shape[0]
    return pl.pallas_call(
        functools.partial(_conv_s2_kernel, nt=nt, oh=oh, relu=relu,
                          out_f32=out_f32),
        grid=(n // nt,),
        in_specs=[
            pl.BlockSpec((nt, 2 * oh, 2 * oh, EMB), lambda i: (i, 0, 0, 0)),
            pl.BlockSpec((9 * EMB, EMB), lambda i: (0, 0)),
            pl.BlockSpec((1, EMB), lambda i: (0, 0)),
        ],
        out_specs=pl.BlockSpec((nt * oh * oh, EMB), lambda i: (i, 0)),
        out_shape=jax.ShapeDtypeStruct((n * oh * oh, EMB),
                                       F32 if out_f32 else BF),
    )(x, w_bf, b.reshape(1, EMB))


def _conv3_kernel(x_ref, w_ref, b_ref, o_ref, *, nt):
    x = x_ref[...]
    acc = b_ref[...].astype(F32)
    for kh in range(3):
        for kw in range(3):
            xs = x[:, kh:kh + 2, kw:kw + 2, :]
            a = xs.reshape(nt * 4, EMB)
            j = kh * 3 + kw
            wj = w_ref[j * EMB:(j + 1) * EMB, :]
            acc = acc + jnp.dot(a, wj, preferred_element_type=F32)
    o_ref[...] = acc


def _conv3(x, w_bf, b, nt=64):
    n = x.shape[0]
    return pl.pallas_call(
        functools.partial(_conv3_kernel, nt=nt),
        grid=(n // nt,),
        in_specs=[
            pl.BlockSpec((nt, 4, 4, EMB), lambda i: (i, 0, 0, 0)),
            pl.BlockSpec((9 * EMB, EMB), lambda i: (0, 0)),
            pl.BlockSpec((1, EMB), lambda i: (0, 0)),
        ],
        out_specs=pl.BlockSpec((nt * 4, EMB), lambda i: (i, 0)),
        out_shape=jax.ShapeDtypeStruct((n * 4, EMB), F32),
    )(x, w_bf, b.reshape(1, EMB))


def _encode(x, w_in_bf, b_in, w1, b1, w2, b2, w3, b3):
    h = _proj(x.reshape(B * S, DIM), w_in_bf, b_in).reshape(B, 16, 16, EMB)
    c1 = _conv_s2(h, w1, b1, oh=8, relu=True, nt=16).reshape(B, 8, 8, EMB)
    c2 = _conv_s2(c1, w2, b2, oh=4, relu=True, nt=64).reshape(B, 4, 4, EMB)
    return _conv3(c2, w3, b3)


def _vq_kernel(zf_ref, zl_ref, cbt_ref, noise_ref, idx_ref, q_ref,
               min_ref, arg_ref, *, kt, nkt):
    k = pl.program_id(0)
    x = zl_ref[...] - zf_ref[...]
    cbt = cbt_ref[...]
    cn = jnp.sum(cbt * cbt, axis=0, keepdims=True)
    scores = cn - 2.0 * jnp.dot(x.astype(BF), cbt.astype(BF),
                                preferred_element_type=F32)
    lmin = jnp.min(scores, axis=1, keepdims=True)
    iota = jax.lax.broadcasted_iota(jnp.int32, scores.shape, 1) + k * kt
    larg = jnp.min(jnp.where(scores == lmin, iota, jnp.int32(2 ** 30)),
                   axis=1, keepdims=True)

    @pl.when(k == 0)
    def _():
        min_ref[...] = lmin
        arg_ref[...] = larg

    @pl.when(k > 0)
    def _():
        better = lmin < min_ref[...]
        min_ref[...] = jnp.where(better, lmin, min_ref[...])
        arg_ref[...] = jnp.where(better, larg, arg_ref[...])

    @pl.when(k == nkt - 1)
    def _():
        xn = jnp.sum(x * x, axis=1, keepdims=True)
        norm_res = jnp.sqrt(jnp.maximum(min_ref[...] + xn, 0.0))
        noise = noise_ref[...]
        norm_rand = jnp.sqrt(jnp.sum(noise * noise, axis=1, keepdims=True))
        q_ref[...] = x + (norm_res / norm_rand + EPS) * noise
        idx_ref[...] = arg_ref[...]


def _vq(zf, zl, cbt, noise, kt=2048):
    nkt = K // kt
    return pl.pallas_call(
        functools.partial(_vq_kernel, kt=kt, nkt=nkt),
        grid=(nkt,),
        in_specs=[
            pl.BlockSpec((R, EMB), lambda k: (0, 0)),
            pl.BlockSpec((R, EMB), lambda k: (0, 0)),
            pl.BlockSpec((EMB, kt), lambda k: (0, k)),
            pl.BlockSpec((R, EMB), lambda k: (0, 0)),
        ],
        out_specs=[
            pl.BlockSpec((R, 1), lambda k: (0, 0)),
            pl.BlockSpec((R, EMB), lambda k: (0, 0)),
        ],
        out_shape=[
            jax.ShapeDtypeStruct((R, 1), jnp.int32),
            jax.ShapeDtypeStruct((R, EMB), F32),
        ],
        scratch_shapes=[
            pltpu.VMEM((R, 1), F32),
            pltpu.VMEM((R, 1), jnp.int32),
        ],
    )(zf, zl, cbt, noise)


def _perp_kernel(idx_ref, o_ref, acc_ref, *, kt, nkt):
    k = pl.program_id(0)
    idx = idx_ref[...]
    iota = jax.lax.broadcasted_iota(jnp.int32, (R, kt), 1) + k * kt
    counts = jnp.sum((idx == iota).astype(F32), axis=0, keepdims=True)
    p = counts / R
    ent = jnp.sum(p * jnp.log(p + 1e-10), axis=1, keepdims=True)

    @pl.when(k == 0)
    def _():
        acc_ref[...] = ent

    @pl.when(k > 0)
    def _():
        acc_ref[...] = acc_ref[...] + ent

    @pl.when(k == nkt - 1)
    def _():
        o_ref[...] = jnp.exp(-acc_ref[...])


def _perplexity(idx, kt=2048):
    nkt = K // kt
    return pl.pallas_call(
        functools.partial(_perp_kernel, kt=kt, nkt=nkt),
        grid=(nkt,),
        in_specs=[pl.BlockSpec((R, 1), lambda k: (0, 0))],
        out_specs=pl.BlockSpec((1, 1), lambda k: (0, 0)),
        out_shape=jax.ShapeDtypeStruct((1, 1), F32),
        scratch_shapes=[pltpu.VMEM((1, 1), F32)],
    )(idx)


def _out_kernel(x_ref, w_ref, b_ref, o_ref):
    acc = jnp.dot(x_ref[...], w_ref[...], preferred_element_type=F32)
    o_ref[...] = acc + b_ref[...]


def _out_mm(q2_bf, w_bf, b):
    return pl.pallas_call(
        _out_kernel,
        grid=(1,),
        in_specs=[
            pl.BlockSpec((R, EMB), lambda i: (0, 0)),
            pl.BlockSpec((EMB, DIM), lambda i: (0, 0)),
            pl.BlockSpec((1, DIM), lambda i: (0, 0)),
        ],
        out_specs=pl.BlockSpec((R, DIM), lambda i: (0, 0)),
        out_shape=jax.ShapeDtypeStruct((R, DIM), F32),
    )(q2_bf, w_bf, b.reshape(1, DIM))


def kernel(input_data_first, input_data_last, codebooks, W_in, b_in,
           conv1_w, conv1_b, conv2_w, conv2_b, conv3_w, conv3_b,
           W_out, b_out, noise):
    w_in_bf = W_in.astype(BF)
    w1 = conv1_w.reshape(9 * EMB, EMB).astype(BF)
    w2 = conv2_w.reshape(9 * EMB, EMB).astype(BF)
    w3 = conv3_w.reshape(9 * EMB, EMB).astype(BF)
    zf = _encode(input_data_first, w_in_bf, b_in, w1, conv1_b, w2, conv2_b,
                 w3, conv3_b)
    zl = _encode(input_data_last, w_in_bf, b_in, w1, conv1_b, w2, conv2_b,
                 w3, conv3_b)
    idx, q = _vq(zf, zl, codebooks.T, noise)
    perp = _perplexity(idx)
    q2 = q.reshape(B, EMB, 4).transpose(0, 2, 1).reshape(R, EMB).astype(BF)
    out = _out_mm(q2, W_out.astype(BF), b_out).reshape(B, 4, DIM)
    return out, perp.reshape(()), idx.reshape(R)
